# baseline (speedup 1.0000x reference)
import jax
import jax.numpy as jnp
from jax.experimental import pallas as pl
from jax.experimental.pallas import tpu as pltpu

_TOTAL = 128 * 128 * 127
_CHUNKS = 8
_SUB, _LN = 2032, 128


def _dma_copy_kernel(a_in, b_in, c_in, d_in, cell_in,
                     a_out, b_out, c_out, d_out, cell_out,
                     buf0, buf1, buf2, buf3, sem_in, sem_out, sem_cell):
    ins = (a_in, b_in, c_in, d_in)
    outs = (a_out, b_out, c_out, d_out)
    bufs = (buf0, buf1, buf2, buf3)

    def in_copy(k, c):
        return pltpu.make_async_copy(ins[k].at[c], bufs[k].at[c % 2],
                                     sem_in.at[c % 2, k])

    def out_copy(k, c):
        return pltpu.make_async_copy(bufs[k].at[c % 2], outs[k].at[c],
                                     sem_out.at[c % 2, k])

    cell_copy = pltpu.make_async_copy(cell_in, cell_out, sem_cell)
    cell_copy.start()
    for k in range(4):
        in_copy(k, 0).start()
    for c in range(_CHUNKS):
        for k in range(4):
            in_copy(k, c).wait()
        for k in range(4):
            out_copy(k, c).start()
        if c + 1 < _CHUNKS:
            if c >= 1:
                for k in range(4):
                    out_copy(k, c - 1).wait()
            for k in range(4):
                in_copy(k, c + 1).start()
    for k in range(4):
        out_copy(k, _CHUNKS - 2).wait()
    for k in range(4):
        out_copy(k, _CHUNKS - 1).wait()
    cell_copy.wait()


def kernel(natoms, pred_distance_displace, pred_var_displace,
           pred_distance_relaxed, pred_var_relaxed, pred_cell):
    any_spec = pl.BlockSpec(memory_space=pl.ANY)
    big_shape = jax.ShapeDtypeStruct((_CHUNKS, _SUB, _LN), jnp.float32)

    a = pred_distance_displace.reshape(_CHUNKS, _SUB, _LN)
    b = pred_var_displace.reshape(_CHUNKS, _SUB, _LN)
    c = pred_distance_relaxed.reshape(_CHUNKS, _SUB, _LN)
    d = pred_var_relaxed.reshape(_CHUNKS, _SUB, _LN)
    cell2d = pred_cell.reshape(128, 9)

    outs = pl.pallas_call(
        _dma_copy_kernel,
        in_specs=[any_spec] * 5,
        out_specs=[any_spec] * 5,
        out_shape=[big_shape] * 4 + [jax.ShapeDtypeStruct((128, 9), jnp.float32)],
        scratch_shapes=[pltpu.VMEM((2, _SUB, _LN), jnp.float32)] * 4
                       + [pltpu.SemaphoreType.DMA((2, 4)),
                          pltpu.SemaphoreType.DMA((2, 4)),
                          pltpu.SemaphoreType.DMA],
    )(a, b, c, d, cell2d)

    n = pred_distance_displace.shape[0]
    return (outs[0].reshape(n), outs[1].reshape(n), outs[2].reshape(n),
            outs[3].reshape(n), outs[4].reshape(128, 3, 3))


# speedup vs baseline: 1.2333x; 1.2333x over previous
import jax
import jax.numpy as jnp
from jax.experimental import pallas as pl
from jax.experimental.pallas import tpu as pltpu

_TOTAL = 128 * 128 * 127
_GRID = 16
_SUB, _LN = _TOTAL // (_GRID * 128), 128


def _copy_kernel(a_in, b_in, c_in, d_in, cell_in,
                 a_out, b_out, c_out, d_out, cell_out):
    a_out[...] = a_in[...]
    b_out[...] = b_in[...]
    c_out[...] = c_in[...]
    d_out[...] = d_in[...]

    @pl.when(pl.program_id(0) == 0)
    def _():
        cell_out[...] = cell_in[...]


def kernel(natoms, pred_distance_displace, pred_var_displace,
           pred_distance_relaxed, pred_var_relaxed, pred_cell):
    big_spec = pl.BlockSpec((1, _SUB, _LN), lambda i: (i, 0, 0))
    cell_spec = pl.BlockSpec((128, 9), lambda i: (0, 0))
    big_shape = jax.ShapeDtypeStruct((_GRID, _SUB, _LN), jnp.float32)

    a = pred_distance_displace.reshape(_GRID, _SUB, _LN)
    b = pred_var_displace.reshape(_GRID, _SUB, _LN)
    c = pred_distance_relaxed.reshape(_GRID, _SUB, _LN)
    d = pred_var_relaxed.reshape(_GRID, _SUB, _LN)
    cell2d = pred_cell.reshape(128, 9)

    outs = pl.pallas_call(
        _copy_kernel,
        grid=(_GRID,),
        in_specs=[big_spec] * 4 + [cell_spec],
        out_specs=[big_spec] * 4 + [cell_spec],
        out_shape=[big_shape] * 4 + [jax.ShapeDtypeStruct((128, 9), jnp.float32)],
    )(a, b, c, d, cell2d)

    n = pred_distance_displace.shape[0]
    return (outs[0].reshape(n), outs[1].reshape(n), outs[2].reshape(n),
            outs[3].reshape(n), outs[4].reshape(128, 3, 3))


# speedup vs baseline: 1.3849x; 1.1229x over previous
import jax
import jax.numpy as jnp
from jax.experimental import pallas as pl
from jax.experimental.pallas import tpu as pltpu

_TOTAL = 128 * 128 * 127
_GRID = 4
_SUB, _LN = _TOTAL // (_GRID * 128), 128


def _copy_kernel(a_in, b_in, c_in, d_in, cell_in,
                 a_out, b_out, c_out, d_out, cell_out):
    a_out[...] = a_in[...]
    b_out[...] = b_in[...]
    c_out[...] = c_in[...]
    d_out[...] = d_in[...]

    @pl.when(pl.program_id(0) == 0)
    def _():
        cell_out[...] = cell_in[...]


def kernel(natoms, pred_distance_displace, pred_var_displace,
           pred_distance_relaxed, pred_var_relaxed, pred_cell):
    big_spec = pl.BlockSpec((1, _SUB, _LN), lambda i: (i, 0, 0))
    cell_spec = pl.BlockSpec((128, 9), lambda i: (0, 0))
    big_shape = jax.ShapeDtypeStruct((_GRID, _SUB, _LN), jnp.float32)

    a = pred_distance_displace.reshape(_GRID, _SUB, _LN)
    b = pred_var_displace.reshape(_GRID, _SUB, _LN)
    c = pred_distance_relaxed.reshape(_GRID, _SUB, _LN)
    d = pred_var_relaxed.reshape(_GRID, _SUB, _LN)
    cell2d = pred_cell.reshape(128, 9)

    outs = pl.pallas_call(
        _copy_kernel,
        grid=(_GRID,),
        in_specs=[big_spec] * 4 + [cell_spec],
        out_specs=[big_spec] * 4 + [cell_spec],
        out_shape=[big_shape] * 4 + [jax.ShapeDtypeStruct((128, 9), jnp.float32)],
    )(a, b, c, d, cell2d)

    n = pred_distance_displace.shape[0]
    return (outs[0].reshape(n), outs[1].reshape(n), outs[2].reshape(n),
            outs[3].reshape(n), outs[4].reshape(128, 3, 3))


# speedup vs baseline: 1.4095x; 1.0178x over previous
import jax
import jax.numpy as jnp
from jax.experimental import pallas as pl
from jax.experimental.pallas import tpu as pltpu

_TOTAL = 128 * 128 * 127
_GRID = 2
_SUB, _LN = _TOTAL // (_GRID * 128), 128


def _copy_kernel(a_in, b_in, c_in, d_in, cell_in,
                 a_out, b_out, c_out, d_out, cell_out):
    a_out[...] = a_in[...]
    b_out[...] = b_in[...]
    c_out[...] = c_in[...]
    d_out[...] = d_in[...]

    @pl.when(pl.program_id(0) == 0)
    def _():
        cell_out[...] = cell_in[...]


def kernel(natoms, pred_distance_displace, pred_var_displace,
           pred_distance_relaxed, pred_var_relaxed, pred_cell):
    big_spec = pl.BlockSpec((1, _SUB, _LN), lambda i: (i, 0, 0))
    cell_spec = pl.BlockSpec((128, 9), lambda i: (0, 0))
    big_shape = jax.ShapeDtypeStruct((_GRID, _SUB, _LN), jnp.float32)

    a = pred_distance_displace.reshape(_GRID, _SUB, _LN)
    b = pred_var_displace.reshape(_GRID, _SUB, _LN)
    c = pred_distance_relaxed.reshape(_GRID, _SUB, _LN)
    d = pred_var_relaxed.reshape(_GRID, _SUB, _LN)
    cell2d = pred_cell.reshape(128, 9)

    outs = pl.pallas_call(
        _copy_kernel,
        grid=(_GRID,),
        compiler_params=pltpu.CompilerParams(vmem_limit_bytes=120*1024*1024),
        in_specs=[big_spec] * 4 + [cell_spec],
        out_specs=[big_spec] * 4 + [cell_spec],
        out_shape=[big_shape] * 4 + [jax.ShapeDtypeStruct((128, 9), jnp.float32)],
    )(a, b, c, d, cell2d)

    n = pred_distance_displace.shape[0]
    return (outs[0].reshape(n), outs[1].reshape(n), outs[2].reshape(n),
            outs[3].reshape(n), outs[4].reshape(128, 3, 3))
